# Initial kernel scaffold; baseline (speedup 1.0000x reference)
import jax
import jax.numpy as jnp
from jax.experimental import pallas as pl


def kernel(x, embed_table, lin_w, lin_b):
    raise NotImplementedError("write your pallas kernel here")


# speedup vs baseline: 85.6310x; 85.6310x over previous
import functools

import jax
import jax.numpy as jnp
from jax import lax
from jax.experimental import pallas as pl
from jax.experimental.pallas import tpu as pltpu
from jax.experimental.pallas import tpu_sc as plsc

B, L = 16384, 200
N = B * L
NC, NS = 2, 16
NW = NC * NS
PER_W = N // NW
CHUNK = 12800
NCHUNK = PER_W // CHUNK
VB = 7
D = 10
VPAD = VB * 16


@functools.partial(
    pl.kernel,
    out_type=jax.ShapeDtypeStruct((N,), jnp.float32),
    mesh=plsc.VectorSubcoreMesh(core_axis_name="c", subcore_axis_name="s"),
    compiler_params=pltpu.CompilerParams(needs_layout_passes=False),
    scratch_types=[
        pltpu.VMEM((D * VPAD,), jnp.float32),
        pltpu.VMEM((D * 16,), jnp.float32),
        pltpu.VMEM((16,), jnp.float32),
        pltpu.VMEM((VPAD,), jnp.float32),
        pltpu.VMEM((CHUNK,), jnp.int32),
        pltpu.VMEM((CHUNK,), jnp.float32),
    ],
)
def _sc_embed(x_hbm, tab_hbm, w_hbm, b_hbm, out_hbm,
              tab_v, w_v, b_v, scores_v, idx_v, out_v):
    wid = lax.axis_index("s") * NC + lax.axis_index("c")

    pltpu.sync_copy(tab_hbm, tab_v)
    pltpu.sync_copy(w_hbm, w_v)
    pltpu.sync_copy(b_hbm, b_v)
    bvec = b_v[...]
    for blk in range(VB):
        acc = bvec
        for d in range(D):
            acc = acc + tab_v[pl.ds(d * VPAD + blk * 16, 16)] * w_v[pl.ds(d * 16, 16)]
        scores_v[pl.ds(blk * 16, 16)] = acc

    base0 = wid * PER_W

    def gather_body(i, carry):
        off = pl.multiple_of(i * 16, 16)
        iv = idx_v[pl.ds(off, 16)]
        out_v[pl.ds(off, 16)] = plsc.load_gather(scores_v, [iv])
        return carry

    for c in range(NCHUNK):
        base = base0 + c * CHUNK
        pltpu.sync_copy(x_hbm.at[pl.ds(base, CHUNK)], idx_v)
        lax.fori_loop(0, CHUNK // 16, gather_body, 0)
        pltpu.sync_copy(out_v, out_hbm.at[pl.ds(base, CHUNK)])


def kernel(x, embed_table, lin_w, lin_b):
    x_flat = x.reshape(N).astype(jnp.int32)
    tab_t = jnp.pad(embed_table.T, ((0, 0), (0, VPAD - embed_table.shape[0])))
    tab_flat = tab_t.reshape(D * VPAD)
    w_b = jnp.repeat(lin_w.reshape(D), 16)
    b_b = jnp.broadcast_to(lin_b.reshape(1), (16,))
    out = _sc_embed(x_flat, tab_flat, w_b, b_b)
    return out.reshape(B, L, 1)


# speedup vs baseline: 118.3940x; 1.3826x over previous
import functools

import jax
import jax.numpy as jnp
from jax import lax
from jax.experimental import pallas as pl
from jax.experimental.pallas import tpu as pltpu
from jax.experimental.pallas import tpu_sc as plsc

B, L = 16384, 200
N = B * L
NC, NS = 2, 16
NW = NC * NS
PER_W = N // NW
CHUNK = 25600
NCHUNK = PER_W // CHUNK
VB = 7
D = 10
VPAD = VB * 16


@functools.partial(
    pl.kernel,
    out_type=jax.ShapeDtypeStruct((N,), jnp.float32),
    mesh=plsc.VectorSubcoreMesh(core_axis_name="c", subcore_axis_name="s"),
    compiler_params=pltpu.CompilerParams(needs_layout_passes=False),
    scratch_types=[
        pltpu.VMEM((D * VPAD,), jnp.float32),
        pltpu.VMEM((D * 16,), jnp.float32),
        pltpu.VMEM((16,), jnp.float32),
        pltpu.VMEM((VPAD,), jnp.float32),
        pltpu.VMEM((CHUNK,), jnp.int32),
        pltpu.VMEM((CHUNK,), jnp.int32),
        pltpu.VMEM((CHUNK,), jnp.float32),
        pltpu.VMEM((CHUNK,), jnp.float32),
        pltpu.SemaphoreType.DMA,
        pltpu.SemaphoreType.DMA,
        pltpu.SemaphoreType.DMA,
        pltpu.SemaphoreType.DMA,
    ],
)
def _sc_embed(x_hbm, tab_hbm, w_hbm, b_hbm, out_hbm,
              tab_v, w_v, b_v, scores_v,
              idx0, idx1, out0, out1, isem0, isem1, osem0, osem1):
    wid = lax.axis_index("s") * NC + lax.axis_index("c")
    base0 = wid * PER_W
    bufs = ((idx0, out0, isem0, osem0), (idx1, out1, isem1, osem1))

    in_copy = [None, None]
    out_copy = [None, None]
    in_copy[0] = pltpu.async_copy(x_hbm.at[pl.ds(base0, CHUNK)], idx0, isem0)

    pltpu.sync_copy(tab_hbm, tab_v)
    pltpu.sync_copy(w_hbm, w_v)
    pltpu.sync_copy(b_hbm, b_v)
    bvec = b_v[...]
    for blk in range(VB):
        acc = bvec
        for d in range(D):
            acc = acc + tab_v[pl.ds(d * VPAD + blk * 16, 16)] * w_v[pl.ds(d * 16, 16)]
        scores_v[pl.ds(blk * 16, 16)] = acc

    for c in range(NCHUNK):
        p = c & 1
        idx_v, out_v, _, osem = bufs[p]
        if c + 1 < NCHUNK:
            q = (c + 1) & 1
            in_copy[q] = pltpu.async_copy(
                x_hbm.at[pl.ds(base0 + (c + 1) * CHUNK, CHUNK)], bufs[q][0], bufs[q][2])
        in_copy[p].wait()
        if out_copy[p] is not None:
            out_copy[p].wait()

        @plsc.parallel_loop(0, CHUNK, step=16, unroll=8)
        def _(i):
            out_v[pl.ds(i, 16)] = plsc.load_gather(scores_v, [idx_v[pl.ds(i, 16)]])

        out_copy[p] = pltpu.async_copy(
            out_v, out_hbm.at[pl.ds(base0 + c * CHUNK, CHUNK)], osem)

    out_copy[(NCHUNK - 2) & 1].wait()
    out_copy[(NCHUNK - 1) & 1].wait()


def kernel(x, embed_table, lin_w, lin_b):
    x_flat = x.reshape(N).astype(jnp.int32)
    tab_t = jnp.pad(embed_table.T, ((0, 0), (0, VPAD - embed_table.shape[0])))
    tab_flat = tab_t.reshape(D * VPAD)
    w_b = jnp.repeat(lin_w.reshape(D), 16)
    b_b = jnp.broadcast_to(lin_b.reshape(1), (16,))
    out = _sc_embed(x_flat, tab_flat, w_b, b_b)
    return out.reshape(B, L, 1)


# speedup vs baseline: 190.4184x; 1.6083x over previous
import functools

import jax
import jax.numpy as jnp
from jax import lax
from jax.experimental import pallas as pl
from jax.experimental.pallas import tpu as pltpu
from jax.experimental.pallas import tpu_sc as plsc

B, L = 16384, 200
NC, NS = 2, 16
NW = NC * NS
ROWS_W = B // NW
RCHUNK = 64
NCHUNK = ROWS_W // RCHUNK
COFFS = tuple(range(0, 192, 16)) + (184,)
VB = 7
D = 10
VPAD = 112
SPAD = 128


@functools.partial(
    pl.kernel,
    out_type=jax.ShapeDtypeStruct((B, L), jnp.float32),
    mesh=plsc.VectorSubcoreMesh(core_axis_name="c", subcore_axis_name="s"),
    compiler_params=pltpu.CompilerParams(needs_layout_passes=False),
    scratch_types=[
        pltpu.VMEM((D * VPAD,), jnp.float32),
        pltpu.VMEM((D * 16,), jnp.float32),
        pltpu.VMEM((16,), jnp.float32),
        pltpu.VMEM((SPAD,), jnp.float32),
        pltpu.VMEM((RCHUNK, L), jnp.int32),
        pltpu.VMEM((RCHUNK, L), jnp.int32),
        pltpu.VMEM((RCHUNK, L), jnp.float32),
        pltpu.VMEM((RCHUNK, L), jnp.float32),
        pltpu.SemaphoreType.DMA,
        pltpu.SemaphoreType.DMA,
        pltpu.SemaphoreType.DMA,
        pltpu.SemaphoreType.DMA,
    ],
)
def _sc_embed(x_hbm, tab_hbm, w_hbm, b_hbm, out_hbm,
              tab_v, w_v, b_v, scores_v,
              idx0, idx1, out0, out1, isem0, isem1, osem0, osem1):
    wid = lax.axis_index("s") * NC + lax.axis_index("c")
    rbase0 = wid * ROWS_W
    bufs = ((idx0, out0, isem0, osem0), (idx1, out1, isem1, osem1))

    in_copy = [None, None]
    out_copy = [None, None]
    in_copy[0] = pltpu.async_copy(
        x_hbm.at[pl.ds(rbase0, RCHUNK), :], idx0, isem0)

    pltpu.sync_copy(tab_hbm, tab_v)
    pltpu.sync_copy(w_hbm, w_v)
    pltpu.sync_copy(b_hbm, b_v)
    bvec = b_v[...]
    zeros = bvec * 0.0
    for blk in range(SPAD // 16):
        if blk < VB:
            acc = bvec
            for d in range(D):
                acc = acc + tab_v[pl.ds(d * VPAD + blk * 16, 16)] * w_v[pl.ds(d * 16, 16)]
        else:
            acc = zeros
        scores_v[pl.ds(blk * 16, 16)] = acc

    for c in range(NCHUNK):
        p = c & 1
        idx_v, out_v, _, osem = bufs[p]
        if c + 1 < NCHUNK:
            q = (c + 1) & 1
            in_copy[q] = pltpu.async_copy(
                x_hbm.at[pl.ds(rbase0 + (c + 1) * RCHUNK, RCHUNK), :],
                bufs[q][0], bufs[q][2])
        in_copy[p].wait()
        if out_copy[p] is not None:
            out_copy[p].wait()

        @plsc.parallel_loop(0, RCHUNK, unroll=2)
        def _(r):
            for co in COFFS:
                iv = lax.bitwise_and(idx_v[r, pl.ds(co, 16)], 127)
                out_v[r, pl.ds(co, 16)] = plsc.load_gather(scores_v, [iv])

        out_copy[p] = pltpu.async_copy(
            out_v, out_hbm.at[pl.ds(rbase0 + c * RCHUNK, RCHUNK), :], osem)

    out_copy[(NCHUNK - 2) & 1].wait()
    out_copy[(NCHUNK - 1) & 1].wait()


def kernel(x, embed_table, lin_w, lin_b):
    tab_t = jnp.pad(embed_table.T, ((0, 0), (0, VPAD - embed_table.shape[0])))
    tab_flat = tab_t.reshape(D * VPAD)
    w_b = jnp.repeat(lin_w.reshape(D), 16)
    b_b = jnp.broadcast_to(lin_b.reshape(1), (16,))
    out = _sc_embed(x.astype(jnp.int32), tab_flat, w_b, b_b)
    return out.reshape(B, L, 1)


# speedup vs baseline: 333.4039x; 1.7509x over previous
import functools

import jax
import jax.numpy as jnp
from jax import lax
from jax.experimental import pallas as pl
from jax.experimental.pallas import tpu as pltpu
from jax.experimental.pallas import tpu_sc as plsc

B, L = 16384, 200
NC, NS = 2, 16
NW = NC * NS
TL, TB = L // 8, B // 128
NCHUNK = TB // NW
VB = 7
D = 10
VPAD = 112


@functools.partial(
    pl.kernel,
    out_type=jax.ShapeDtypeStruct((TL, 8, B), jnp.float32),
    mesh=plsc.VectorSubcoreMesh(core_axis_name="c", subcore_axis_name="s"),
    compiler_params=pltpu.CompilerParams(
        needs_layout_passes=False, use_tc_tiling_on_sc=False),
    scratch_types=[
        pltpu.VMEM((D * VPAD,), jnp.float32),
        pltpu.VMEM((D * 16,), jnp.float32),
        pltpu.VMEM((16,), jnp.float32),
        pltpu.VMEM((VPAD,), jnp.float32),
        pltpu.VMEM((TL, 8, 128), jnp.int32),
        pltpu.VMEM((TL, 8, 128), jnp.int32),
        pltpu.VMEM((TL, 8, 128), jnp.float32),
        pltpu.VMEM((TL, 8, 128), jnp.float32),
        pltpu.SemaphoreType.DMA,
        pltpu.SemaphoreType.DMA,
        pltpu.SemaphoreType.DMA,
        pltpu.SemaphoreType.DMA,
    ],
)
def _sc_embed(x4_hbm, tab_hbm, w_hbm, b_hbm, out_hbm,
              tab_v, w_v, b_v, scores_v,
              idx0, idx1, out0, out1, isem0, isem1, osem0, osem1):
    wid = lax.axis_index("s") * NC + lax.axis_index("c")
    tc0 = wid * NCHUNK
    bufs = ((idx0, out0, isem0, osem0), (idx1, out1, isem1, osem1))

    in_copy = [None, None]
    out_copy = [None, None]
    in_copy[0] = pltpu.async_copy(x4_hbm.at[:, tc0], idx0, isem0)

    pltpu.sync_copy(tab_hbm, tab_v)
    pltpu.sync_copy(w_hbm, w_v)
    pltpu.sync_copy(b_hbm, b_v)
    bvec = b_v[...]
    for blk in range(VB):
        acc = bvec
        for d in range(D):
            acc = acc + tab_v[pl.ds(d * VPAD + blk * 16, 16)] * w_v[pl.ds(d * 16, 16)]
        scores_v[pl.ds(blk * 16, 16)] = acc

    for c in range(NCHUNK):
        p = c & 1
        idx_v, out_v, _, osem = bufs[p]
        if c + 1 < NCHUNK:
            q = (c + 1) & 1
            in_copy[q] = pltpu.async_copy(
                x4_hbm.at[:, tc0 + c + 1], bufs[q][0], bufs[q][2])
        in_copy[p].wait()
        if out_copy[p] is not None:
            out_copy[p].wait()

        @plsc.parallel_loop(0, TL, unroll=1)
        def _(a):
            for s in range(8):
                for co in range(0, 128, 16):
                    iv = idx_v[a, s, pl.ds(co, 16)]
                    out_v[a, s, pl.ds(co, 16)] = plsc.load_gather(scores_v, [iv])

        out_copy[p] = pltpu.async_copy(
            out_v, out_hbm.at[:, :, pl.ds((tc0 + c) * 128, 128)], osem)

    out_copy[(NCHUNK - 2) & 1].wait()
    out_copy[(NCHUNK - 1) & 1].wait()


def kernel(x, embed_table, lin_w, lin_b):
    tab_t = jnp.pad(embed_table.T, ((0, 0), (0, VPAD - embed_table.shape[0])))
    tab_flat = tab_t.reshape(D * VPAD)
    w_b = jnp.repeat(lin_w.reshape(D), 16)
    b_b = jnp.broadcast_to(lin_b.reshape(1), (16,))
    x4 = jnp.swapaxes(x, 0, 1).astype(jnp.int32) \
        .reshape(TL, 8, TB, 128).transpose(0, 2, 1, 3)
    out = _sc_embed(x4, tab_flat, w_b, b_b)
    return out.reshape(1, L, B).transpose(2, 1, 0)


# speedup vs baseline: 335.5513x; 1.0064x over previous
import functools

import jax
import jax.numpy as jnp
from jax import lax
from jax.experimental import pallas as pl
from jax.experimental.pallas import tpu as pltpu
from jax.experimental.pallas import tpu_sc as plsc

B, L = 16384, 200
NC, NS = 2, 16
NW = NC * NS
TL, TB = L // 8, B // 128
NCHUNK = TB // NW
VB = 7
D = 10
VPAD = 112


@functools.partial(
    pl.kernel,
    out_type=jax.ShapeDtypeStruct((TL, 8, B), jnp.float32),
    mesh=plsc.VectorSubcoreMesh(core_axis_name="c", subcore_axis_name="s"),
    compiler_params=pltpu.CompilerParams(
        needs_layout_passes=False, use_tc_tiling_on_sc=False),
    scratch_types=[
        pltpu.VMEM((100 * D,), jnp.float32),
        pltpu.VMEM((D + 1,), jnp.float32),
        pltpu.VMEM((VPAD,), jnp.float32),
        pltpu.VMEM((TL, 8, 128), jnp.int32),
        pltpu.VMEM((TL, 8, 128), jnp.int32),
        pltpu.VMEM((TL, 8, 128), jnp.float32),
        pltpu.VMEM((TL, 8, 128), jnp.float32),
        pltpu.SemaphoreType.DMA,
        pltpu.SemaphoreType.DMA,
        pltpu.SemaphoreType.DMA,
        pltpu.SemaphoreType.DMA,
    ],
)
def _sc_embed(x4_hbm, tab_hbm, wb_hbm, out_hbm,
              tab_v, wb_v, scores_v,
              idx0, idx1, out0, out1, isem0, isem1, osem0, osem1):
    wid = lax.axis_index("s") * NC + lax.axis_index("c")
    tc0 = wid * NCHUNK
    bufs = ((idx0, out0, isem0, osem0), (idx1, out1, isem1, osem1))

    in_copy = [None, None]
    out_copy = [None, None]
    in_copy[0] = pltpu.async_copy(x4_hbm.at[:, tc0], idx0, isem0)

    pltpu.sync_copy(tab_hbm, tab_v)
    pltpu.sync_copy(wb_hbm, wb_v)
    lane = lax.iota(jnp.int32, 16)
    bvec = plsc.load_gather(wb_v, [jnp.full((16,), D, jnp.int32)])
    for blk in range(VB):
        acc = bvec
        vidx = jnp.minimum(lane + blk * 16, 99)
        for d in range(D):
            dvec = jnp.full((16,), d, jnp.int32)
            acc = acc + (plsc.load_gather(tab_v, [vidx + d * 100])
                         * plsc.load_gather(wb_v, [dvec]))
        scores_v[pl.ds(blk * 16, 16)] = acc

    for c in range(NCHUNK):
        p = c & 1
        idx_v, out_v, _, osem = bufs[p]
        if c + 1 < NCHUNK:
            q = (c + 1) & 1
            in_copy[q] = pltpu.async_copy(
                x4_hbm.at[:, tc0 + c + 1], bufs[q][0], bufs[q][2])
        in_copy[p].wait()
        if out_copy[p] is not None:
            out_copy[p].wait()

        @plsc.parallel_loop(0, TL, unroll=1)
        def _(a):
            for s in range(8):
                for co in range(0, 128, 16):
                    iv = idx_v[a, s, pl.ds(co, 16)]
                    out_v[a, s, pl.ds(co, 16)] = plsc.load_gather(scores_v, [iv])

        out_copy[p] = pltpu.async_copy(
            out_v, out_hbm.at[:, :, pl.ds((tc0 + c) * 128, 128)], osem)

    out_copy[(NCHUNK - 2) & 1].wait()
    out_copy[(NCHUNK - 1) & 1].wait()


def kernel(x, embed_table, lin_w, lin_b):
    wb = jnp.concatenate([lin_w.reshape(D), lin_b])
    x4 = jnp.swapaxes(x, 0, 1).astype(jnp.int32) \
        .reshape(TL, 8, TB, 128).transpose(0, 2, 1, 3)
    out = _sc_embed(x4, embed_table.T.reshape(100 * D), wb)
    return out.reshape(1, L, B).transpose(2, 1, 0)


# speedup vs baseline: 336.7509x; 1.0036x over previous
import functools

import jax
import jax.numpy as jnp
from jax import lax
from jax.experimental import pallas as pl
from jax.experimental.pallas import tpu as pltpu
from jax.experimental.pallas import tpu_sc as plsc

B, L = 16384, 200
NC, NS = 2, 16
NW = NC * NS
TL, TB = L // 8, B // 128
NCHUNK = TB // NW
VB = 7
D = 10
VPAD = 112


@functools.partial(
    pl.kernel,
    out_type=jax.ShapeDtypeStruct((TL, 8, B), jnp.float32),
    mesh=plsc.VectorSubcoreMesh(core_axis_name="c", subcore_axis_name="s"),
    compiler_params=pltpu.CompilerParams(
        needs_layout_passes=False, use_tc_tiling_on_sc=False),
    scratch_types=[
        pltpu.VMEM((1024,), jnp.float32),
        pltpu.VMEM((VPAD,), jnp.float32),
        pltpu.VMEM((TL, 8, 128), jnp.int32),
        pltpu.VMEM((TL, 8, 128), jnp.int32),
        pltpu.VMEM((TL, 8, 128), jnp.float32),
        pltpu.VMEM((TL, 8, 128), jnp.float32),
        pltpu.SemaphoreType.DMA,
        pltpu.SemaphoreType.DMA,
        pltpu.SemaphoreType.DMA,
        pltpu.SemaphoreType.DMA,
    ],
)
def _sc_embed(x4_hbm, wts_hbm, out_hbm,
              wts_v, scores_v,
              idx0, idx1, out0, out1, isem0, isem1, osem0, osem1):
    wid = lax.axis_index("s") * NC + lax.axis_index("c")
    tc0 = wid * NCHUNK
    bufs = ((idx0, out0, isem0, osem0), (idx1, out1, isem1, osem1))

    in_copy = [None, None]
    out_copy = [None, None]
    in_copy[0] = pltpu.async_copy(x4_hbm.at[:, tc0], idx0, isem0)

    pltpu.sync_copy(wts_hbm, wts_v)
    lane = lax.iota(jnp.int32, 16)
    bvec = plsc.load_gather(wts_v, [jnp.full((16,), 1010, jnp.int32)])
    for blk in range(VB):
        acc = bvec
        vidx = jnp.minimum(lane + blk * 16, 99)
        for d in range(D):
            dvec = jnp.full((16,), 1000 + d, jnp.int32)
            acc = acc + (plsc.load_gather(wts_v, [vidx + d * 100])
                         * plsc.load_gather(wts_v, [dvec]))
        scores_v[pl.ds(blk * 16, 16)] = acc

    for c in range(NCHUNK):
        p = c & 1
        idx_v, out_v, _, osem = bufs[p]
        if c + 1 < NCHUNK:
            q = (c + 1) & 1
            in_copy[q] = pltpu.async_copy(
                x4_hbm.at[:, tc0 + c + 1], bufs[q][0], bufs[q][2])
        in_copy[p].wait()
        if out_copy[p] is not None:
            out_copy[p].wait()

        @plsc.parallel_loop(0, TL, unroll=1)
        def _(a):
            for s in range(8):
                for co in range(0, 128, 16):
                    iv = idx_v[a, s, pl.ds(co, 16)]
                    out_v[a, s, pl.ds(co, 16)] = plsc.load_gather(scores_v, [iv])

        out_copy[p] = pltpu.async_copy(
            out_v, out_hbm.at[:, :, pl.ds((tc0 + c) * 128, 128)], osem)

    out_copy[(NCHUNK - 2) & 1].wait()
    out_copy[(NCHUNK - 1) & 1].wait()


def kernel(x, embed_table, lin_w, lin_b):
    wts = jnp.concatenate([embed_table.T.reshape(100 * D),
                           lin_w.reshape(D), lin_b,
                           jnp.zeros((13,), jnp.float32)])
    x4 = jnp.swapaxes(x, 0, 1).astype(jnp.int32) \
        .reshape(TL, 8, TB, 128).transpose(0, 2, 1, 3)
    out = _sc_embed(x4, wts)
    return out.reshape(1, L, B).transpose(2, 1, 0)


# speedup vs baseline: 384.3252x; 1.1413x over previous
import functools

import jax
import jax.numpy as jnp
from jax import lax
from jax.experimental import pallas as pl
from jax.experimental.pallas import tpu as pltpu
from jax.experimental.pallas import tpu_sc as plsc

B, L = 16384, 200
NC, NS = 2, 16
NW = NC * NS
TL, TB = L // 8, B // 128
NCHUNK = TB // NW
VB = 7
D = 10
VPAD = 112


@functools.partial(
    pl.kernel,
    out_type=jax.ShapeDtypeStruct((TL, 8, B), jnp.float32),
    mesh=plsc.VectorSubcoreMesh(core_axis_name="c", subcore_axis_name="s"),
    compiler_params=pltpu.CompilerParams(
        needs_layout_passes=False, use_tc_tiling_on_sc=False),
    scratch_types=[
        pltpu.VMEM((1024,), jnp.float32),
        pltpu.VMEM((VPAD,), jnp.float32),
        pltpu.VMEM((TL, 8, 128), jnp.int32),
        pltpu.VMEM((TL, 8, 128), jnp.int32),
        pltpu.VMEM((TL, 8, 128), jnp.float32),
        pltpu.VMEM((TL, 8, 128), jnp.float32),
        pltpu.SemaphoreType.DMA,
        pltpu.SemaphoreType.DMA,
        pltpu.SemaphoreType.DMA,
        pltpu.SemaphoreType.DMA,
    ],
)
def _sc_embed(x4_hbm, wts_hbm, out_hbm,
              wts_v, scores_v,
              idx0, idx1, out0, out1, isem0, isem1, osem0, osem1):
    wid = lax.axis_index("s") * NC + lax.axis_index("c")
    tc0 = wid * NCHUNK
    bufs = ((idx0, out0, isem0, osem0), (idx1, out1, isem1, osem1))

    in_copy = [None, None]
    out_copy = [None, None]
    in_copy[0] = pltpu.async_copy(x4_hbm.at[:, tc0], idx0, isem0)

    pltpu.sync_copy(wts_hbm, wts_v)
    lane = lax.iota(jnp.int32, 16)
    bvec = plsc.load_gather(wts_v, [jnp.full((16,), 1010, jnp.int32)])
    for blk in range(VB):
        acc = bvec
        vidx = jnp.minimum(lane + blk * 16, 99)
        for d in range(D):
            dvec = jnp.full((16,), 1000 + d, jnp.int32)
            acc = acc + (plsc.load_gather(wts_v, [vidx + d * 100])
                         * plsc.load_gather(wts_v, [dvec]))
        scores_v[pl.ds(blk * 16, 16)] = acc

    for c in range(NCHUNK):
        p = c & 1
        idx_v, out_v, _, osem = bufs[p]
        if c + 1 < NCHUNK:
            q = (c + 1) & 1
            in_copy[q] = pltpu.async_copy(
                x4_hbm.at[:, tc0 + c + 1], bufs[q][0], bufs[q][2])
        in_copy[p].wait()
        if out_copy[p] is not None:
            out_copy[p].wait()

        @plsc.parallel_loop(0, L, unroll=2)
        def _(r):
            a = r // 8
            s = r % 8
            for co in range(0, 128, 16):
                iv = idx_v[a, s, pl.ds(co, 16)]
                out_v[a, s, pl.ds(co, 16)] = plsc.load_gather(scores_v, [iv])

        out_copy[p] = pltpu.async_copy(
            out_v, out_hbm.at[:, :, pl.ds((tc0 + c) * 128, 128)], osem)

    out_copy[(NCHUNK - 2) & 1].wait()
    out_copy[(NCHUNK - 1) & 1].wait()


def kernel(x, embed_table, lin_w, lin_b):
    wts = jnp.concatenate([embed_table.T.reshape(100 * D),
                           lin_w.reshape(D), lin_b,
                           jnp.zeros((13,), jnp.float32)])
    x4 = jnp.swapaxes(x, 0, 1).astype(jnp.int32) \
        .reshape(TL, 8, TB, 128).transpose(0, 2, 1, 3)
    out = _sc_embed(x4, wts)
    return out.reshape(1, L, B).transpose(2, 1, 0)


# speedup vs baseline: 388.9936x; 1.0121x over previous
import functools

import jax
import jax.numpy as jnp
from jax import lax
from jax.experimental import pallas as pl
from jax.experimental.pallas import tpu as pltpu
from jax.experimental.pallas import tpu_sc as plsc

B, L = 16384, 200
NC, NS = 2, 16
NW = NC * NS
TL, TB = L // 8, B // 128
NCHUNK = TB // NW
VB = 7
D = 10
VPAD = 112


@functools.partial(
    pl.kernel,
    out_type=jax.ShapeDtypeStruct((TL, 8, B), jnp.float32),
    mesh=plsc.VectorSubcoreMesh(core_axis_name="c", subcore_axis_name="s"),
    compiler_params=pltpu.CompilerParams(
        needs_layout_passes=False, use_tc_tiling_on_sc=False),
    scratch_types=[
        pltpu.VMEM((1024,), jnp.float32),
        pltpu.VMEM((VPAD,), jnp.float32),
        pltpu.VMEM((TL, 8, 128), jnp.int32),
        pltpu.VMEM((TL, 8, 128), jnp.int32),
        pltpu.VMEM((TL, 8, 128), jnp.float32),
        pltpu.VMEM((TL, 8, 128), jnp.float32),
        pltpu.SemaphoreType.DMA,
        pltpu.SemaphoreType.DMA,
        pltpu.SemaphoreType.DMA,
        pltpu.SemaphoreType.DMA,
    ],
)
def _sc_embed(x4_hbm, wts_hbm, out_hbm,
              wts_v, scores_v,
              idx0, idx1, out0, out1, isem0, isem1, osem0, osem1):
    wid = lax.axis_index("s") * NC + lax.axis_index("c")
    tc0 = wid * NCHUNK
    bufs = ((idx0, out0, isem0, osem0), (idx1, out1, isem1, osem1))

    in_copy = [None, None]
    out_copy = [None, None]
    in_copy[0] = pltpu.async_copy(x4_hbm.at[:, tc0], idx0, isem0)

    pltpu.sync_copy(wts_hbm, wts_v)
    lane = lax.iota(jnp.int32, 16)
    bvec = plsc.load_gather(wts_v, [jnp.full((16,), 1010, jnp.int32)])

    @plsc.parallel_loop(0, VB)
    def _(blk):
        acc = bvec
        vidx = jnp.minimum(lane + blk * 16, 99)
        for d in range(D):
            dvec = jnp.full((16,), 1000 + d, jnp.int32)
            acc = acc + (plsc.load_gather(wts_v, [vidx + d * 100])
                         * plsc.load_gather(wts_v, [dvec]))
        scores_v[pl.ds(pl.multiple_of(blk * 16, 16), 16)] = acc

    for c in range(NCHUNK):
        p = c & 1
        idx_v, out_v, _, osem = bufs[p]
        if c + 1 < NCHUNK:
            q = (c + 1) & 1
            in_copy[q] = pltpu.async_copy(
                x4_hbm.at[:, tc0 + c + 1], bufs[q][0], bufs[q][2])
        in_copy[p].wait()
        if out_copy[p] is not None:
            out_copy[p].wait()

        @plsc.parallel_loop(0, L, unroll=2)
        def _(r):
            a = r // 8
            s = r % 8
            for co in range(0, 128, 16):
                iv = idx_v[a, s, pl.ds(co, 16)]
                out_v[a, s, pl.ds(co, 16)] = plsc.load_gather(scores_v, [iv])

        out_copy[p] = pltpu.async_copy(
            out_v, out_hbm.at[:, :, pl.ds((tc0 + c) * 128, 128)], osem)

    out_copy[(NCHUNK - 2) & 1].wait()
    out_copy[(NCHUNK - 1) & 1].wait()


def kernel(x, embed_table, lin_w, lin_b):
    wts = jnp.concatenate([embed_table.T.reshape(100 * D),
                           lin_w.reshape(D), lin_b,
                           jnp.zeros((13,), jnp.float32)])
    x4 = jnp.swapaxes(x, 0, 1).astype(jnp.int32) \
        .reshape(TL, 8, TB, 128).transpose(0, 2, 1, 3)
    out = _sc_embed(x4, wts)
    return out.reshape(1, L, B).transpose(2, 1, 0)


# speedup vs baseline: 395.9251x; 1.0178x over previous
import functools

import jax
import jax.numpy as jnp
from jax import lax
from jax.experimental import pallas as pl
from jax.experimental.pallas import tpu as pltpu
from jax.experimental.pallas import tpu_sc as plsc

B, L = 16384, 200
NC, NS = 2, 16
NW = NC * NS
TL, TB = L // 8, B // 128
NCHUNK = TB // NW
VB = 7
D = 10
VPAD = 112


@functools.partial(
    pl.kernel,
    out_type=jax.ShapeDtypeStruct((TL, 8, B), jnp.float32),
    mesh=plsc.VectorSubcoreMesh(core_axis_name="c", subcore_axis_name="s"),
    compiler_params=pltpu.CompilerParams(
        needs_layout_passes=False, use_tc_tiling_on_sc=False),
    scratch_types=[
        pltpu.VMEM((1024,), jnp.float32),
        pltpu.VMEM((VPAD,), jnp.float32),
        pltpu.VMEM((TL, 8, 128), jnp.int32),
        pltpu.VMEM((TL, 8, 128), jnp.int32),
        pltpu.VMEM((TL, 8, 128), jnp.float32),
        pltpu.VMEM((TL, 8, 128), jnp.float32),
        pltpu.SemaphoreType.DMA,
        pltpu.SemaphoreType.DMA,
        pltpu.SemaphoreType.DMA,
        pltpu.SemaphoreType.DMA,
    ],
)
def _sc_embed(x4_hbm, wts_hbm, out_hbm,
              wts_v, scores_v,
              idx0, idx1, out0, out1, isem0, isem1, osem0, osem1):
    wid = lax.axis_index("s") * NC + lax.axis_index("c")
    tc0 = wid * NCHUNK
    bufs = ((idx0, out0, isem0, osem0), (idx1, out1, isem1, osem1))

    in_copy = [None, None]
    out_copy = [None, None]
    in_copy[0] = pltpu.async_copy(x4_hbm.at[:, tc0], idx0, isem0)

    pltpu.sync_copy(wts_hbm, wts_v)
    lane = lax.iota(jnp.int32, 16)
    bvec = plsc.load_gather(wts_v, [jnp.full((16,), 1010, jnp.int32)])

    @plsc.parallel_loop(0, VB)
    def _(blk):
        acc = bvec
        vidx = jnp.minimum(lane + blk * 16, 99)
        for d in range(D):
            dvec = jnp.full((16,), 1000 + d, jnp.int32)
            acc = acc + (plsc.load_gather(wts_v, [vidx + d * 100])
                         * plsc.load_gather(wts_v, [dvec]))
        scores_v[pl.ds(pl.multiple_of(blk * 16, 16), 16)] = acc

    for c in range(NCHUNK):
        p = c & 1
        idx_v, out_v, _, osem = bufs[p]
        if c + 1 < NCHUNK:
            q = (c + 1) & 1
            in_copy[q] = pltpu.async_copy(
                x4_hbm.at[:, tc0 + c + 1], bufs[q][0], bufs[q][2])
        in_copy[p].wait()
        if out_copy[p] is not None:
            out_copy[p].wait()

        @plsc.parallel_loop(0, L, unroll=1)
        def _(r):
            a = r // 8
            s = r % 8
            for co in range(0, 128, 16):
                iv = idx_v[a, s, pl.ds(co, 16)]
                out_v[a, s, pl.ds(co, 16)] = plsc.load_gather(scores_v, [iv])

        out_copy[p] = pltpu.async_copy(
            out_v, out_hbm.at[:, :, pl.ds((tc0 + c) * 128, 128)], osem)

    out_copy[(NCHUNK - 2) & 1].wait()
    out_copy[(NCHUNK - 1) & 1].wait()


def kernel(x, embed_table, lin_w, lin_b):
    wts = jnp.concatenate([embed_table.T.reshape(100 * D),
                           lin_w.reshape(D), lin_b,
                           jnp.zeros((13,), jnp.float32)])
    x4 = jnp.swapaxes(x, 0, 1).astype(jnp.int32) \
        .reshape(TL, 8, TB, 128).transpose(0, 2, 1, 3)
    out = _sc_embed(x4, wts)
    return out.reshape(1, L, B).transpose(2, 1, 0)


# speedup vs baseline: 399.7865x; 1.0098x over previous
import functools

import jax
import jax.numpy as jnp
from jax import lax
from jax.experimental import pallas as pl
from jax.experimental.pallas import tpu as pltpu
from jax.experimental.pallas import tpu_sc as plsc

B, L = 16384, 200
NC, NS = 2, 16
NW = NC * NS
TL, TB = L // 8, B // 128
NCHUNK = TB // NW
VB = 7
D = 10
VPAD = 112


@functools.partial(
    pl.kernel,
    out_type=jax.ShapeDtypeStruct((TL, 8, B), jnp.float32),
    mesh=plsc.VectorSubcoreMesh(core_axis_name="c", subcore_axis_name="s"),
    compiler_params=pltpu.CompilerParams(
        needs_layout_passes=False, use_tc_tiling_on_sc=False),
    scratch_types=[
        pltpu.VMEM((1024,), jnp.float32),
        pltpu.VMEM((VPAD,), jnp.float32),
        pltpu.VMEM((TL, 8, 128), jnp.int32),
        pltpu.VMEM((TL, 8, 128), jnp.int32),
        pltpu.VMEM((TL, 8, 128), jnp.float32),
        pltpu.VMEM((TL, 8, 128), jnp.float32),
        pltpu.SemaphoreType.DMA,
        pltpu.SemaphoreType.DMA,
        pltpu.SemaphoreType.DMA,
        pltpu.SemaphoreType.DMA,
    ],
)
def _sc_embed(x4_hbm, wts_hbm, out_hbm,
              wts_v, scores_v,
              idx0, idx1, out0, out1, isem0, isem1, osem0, osem1):
    wid = lax.axis_index("s") * NC + lax.axis_index("c")
    tc0 = wid * NCHUNK
    bufs = ((idx0, out0, isem0, osem0), (idx1, out1, isem1, osem1))

    pltpu.async_copy(x4_hbm.at[:, tc0], idx0, isem0)

    pltpu.sync_copy(wts_hbm, wts_v)
    lane = lax.iota(jnp.int32, 16)
    bvec = plsc.load_gather(wts_v, [jnp.full((16,), 1010, jnp.int32)])

    @plsc.parallel_loop(0, VB)
    def _(blk):
        acc = bvec
        vidx = jnp.minimum(lane + blk * 16, 99)
        for d in range(D):
            dvec = jnp.full((16,), 1000 + d, jnp.int32)
            acc = acc + (plsc.load_gather(wts_v, [vidx + d * 100])
                         * plsc.load_gather(wts_v, [dvec]))
        scores_v[pl.ds(pl.multiple_of(blk * 16, 16), 16)] = acc

    pltpu.async_copy(x4_hbm.at[:, tc0 + 1], idx1, isem1)

    def ring_body(g, carry):
        for b in range(2):
            idx_v, out_v, isem, osem = bufs[b]
            c = 2 * g + b
            pltpu.make_async_copy(x4_hbm.at[:, tc0], idx_v, isem).wait()

            @pl.when(g > 0)
            def _():
                pltpu.make_async_copy(
                    out_v, out_hbm.at[:, :, pl.ds(0, 128)], osem).wait()

            @plsc.parallel_loop(0, L, unroll=1)
            def _(r):
                a = r // 8
                s = r % 8
                for co in range(0, 128, 16):
                    iv = idx_v[a, s, pl.ds(co, 16)]
                    out_v[a, s, pl.ds(co, 16)] = plsc.load_gather(scores_v, [iv])

            pltpu.async_copy(
                out_v, out_hbm.at[:, :, pl.ds((tc0 + c) * 128, 128)], osem)

            @pl.when(c + 2 < NCHUNK)
            def _():
                pltpu.async_copy(x4_hbm.at[:, tc0 + c + 2], idx_v, isem)
        return carry

    lax.fori_loop(0, NCHUNK // 2, ring_body, 0)
    for b in range(2):
        pltpu.make_async_copy(
            bufs[b][1], out_hbm.at[:, :, pl.ds(0, 128)], bufs[b][3]).wait()


def kernel(x, embed_table, lin_w, lin_b):
    wts = jnp.concatenate([embed_table.T.reshape(100 * D),
                           lin_w.reshape(D), lin_b,
                           jnp.zeros((13,), jnp.float32)])
    x4 = jnp.swapaxes(x, 0, 1).astype(jnp.int32) \
        .reshape(TL, 8, TB, 128).transpose(0, 2, 1, 3)
    out = _sc_embed(x4, wts)
    return out.reshape(1, L, B).transpose(2, 1, 0)
